# baseline (speedup 1.0000x reference)
import functools

import jax
import jax.numpy as jnp
from jax import lax
from jax.experimental import pallas as pl
from jax.experimental.pallas import tpu as pltpu
from jax.experimental.pallas import tpu_sc as plsc

N = 10000
E = 160000
IN_FEATS = 256
HALF = 128
HIDDEN = 512
NUM_CLASSES = 128

CHUNK = 128
ROWS = E // CHUNK
NSUB = 16
LANES = 16
DEG_W = 128
ROWS_PC = ROWS // 2
RPW = (N // NSUB) // 8 * 8
LEFT_OUT = N - NSUB * RPW

_MESH = plsc.VectorSubcoreMesh(core_axis_name="c", subcore_axis_name="s")


GRP = 8
NGRP = ROWS // GRP
LEFT_ROWS = ROWS - NGRP * GRP
NB = 2
DEPTH = 2


def _sc_sum(xs, src2, dst2, zsum):
    @functools.partial(
        pl.kernel,
        out_type=jax.ShapeDtypeStruct((2 * N, HALF), jnp.float32),
        mesh=_MESH,
        scratch_types=[
            pltpu.VMEM((GRP, CHUNK), jnp.int32),
            pltpu.VMEM((GRP, CHUNK), jnp.int32),
            pltpu.VMEM((CHUNK, HALF), jnp.float32),
            pltpu.VMEM((CHUNK, HALF), jnp.float32),
            pltpu.SemaphoreType.DMA,
            pltpu.SemaphoreType.DMA,
            pltpu.SemaphoreType.DMA,
            pltpu.SemaphoreType.DMA,
            pltpu.VMEM_SHARED((N, HALF), jnp.float32),
        ],
    )
    def agg(xs_hbm, src_hbm, dst_hbm, zsum_hbm, out_hbm,
            src_g, dst_g, b0, b1, g0, g1, s0, s1, ssum):
        cid = lax.axis_index("c")
        sid = lax.axis_index("s")
        base = sid * RPW
        bufs = (b0, b1)
        gsem = (g0, g1)
        ssem = (s0, s1)

        pltpu.sync_copy(zsum_hbm.at[pl.ds(base, RPW)],
                        ssum.at[pl.ds(base, RPW)])

        @pl.when(sid == NSUB - 1)
        def _():
            pltpu.sync_copy(zsum_hbm.at[pl.ds(NSUB * RPW, LEFT_OUT)],
                            ssum.at[pl.ds(NSUB * RPW, LEFT_OUT)])

        plsc.subcore_barrier()

        off = cid

        def do_group(row0, nrows):
            pltpu.sync_copy(src_hbm.at[pl.ds(row0, nrows)],
                            src_g.at[pl.ds(0, nrows)])
            pltpu.sync_copy(dst_hbm.at[pl.ds(row0, nrows)],
                            dst_g.at[pl.ds(0, nrows)])
            for j in range(nrows):
                for k in range(CHUNK // LANES):
                    sl = pl.ds(k * LANES, LANES)
                    src_g[j, sl] = src_g[j, sl] + off

            gh = {}
            sh = {}
            waited = set()
            for j in range(min(DEPTH, nrows)):
                b = j % NB
                gh[j] = pltpu.async_copy(xs_hbm.at[src_g.at[j]],
                                         bufs[b], gsem[b])
            for j in range(nrows):
                b = j % NB
                gh[j].wait()
                sh[j] = pltpu.async_copy(bufs[b], ssum.at[dst_g.at[j]],
                                         ssem[b], add=True)
                nj = j + DEPTH
                if nj < nrows:
                    nb_ = nj % NB
                    if nj - NB >= 0:
                        sh[nj - NB].wait()
                        waited.add(nj - NB)
                    gh[nj] = pltpu.async_copy(xs_hbm.at[src_g.at[nj]],
                                              bufs[nb_], gsem[nb_])
            for j in range(nrows):
                if j not in waited:
                    sh[j].wait()

        @pl.loop(0, (NGRP + NSUB - 1) // NSUB)
        def _(i):
            g = sid + i * NSUB

            @pl.when(g < NGRP)
            def _():
                do_group(g * GRP, GRP)

        @pl.when(sid == NSUB - 1)
        def _():
            do_group(NGRP * GRP, LEFT_ROWS)

        plsc.subcore_barrier()

        out_off = cid * N
        pltpu.sync_copy(ssum.at[pl.ds(base, RPW)],
                        out_hbm.at[pl.ds(out_off + base, RPW)])

        @pl.when(sid == NSUB - 1)
        def _():
            pltpu.sync_copy(ssum.at[pl.ds(NSUB * RPW, LEFT_OUT)],
                            out_hbm.at[pl.ds(out_off + NSUB * RPW, LEFT_OUT)])

    return agg(xs, src2, dst2, zsum)


DGRP_PC = (ROWS // 2) // GRP
DLEFT = ROWS - 2 * DGRP_PC * GRP


def _sc_deg(dst2, zdeg, ones):
    @functools.partial(
        pl.kernel,
        out_type=jax.ShapeDtypeStruct((2 * N, DEG_W), jnp.float32),
        mesh=_MESH,
        scratch_types=[
            pltpu.VMEM((GRP, CHUNK), jnp.int32),
            pltpu.VMEM((CHUNK, DEG_W), jnp.float32),
            pltpu.SemaphoreType.DMA,
            pltpu.VMEM_SHARED((N, DEG_W), jnp.float32),
        ],
    )
    def agg(dst_hbm, zdeg_hbm, ones_hbm, out_hbm, dst_g, ones_v, ssem, sdeg):
        cid = lax.axis_index("c")
        sid = lax.axis_index("s")
        base = sid * RPW

        pltpu.sync_copy(zdeg_hbm.at[pl.ds(base, RPW)],
                        sdeg.at[pl.ds(base, RPW)])

        @pl.when(sid == NSUB - 1)
        def _():
            pltpu.sync_copy(zdeg_hbm.at[pl.ds(NSUB * RPW, LEFT_OUT)],
                            sdeg.at[pl.ds(NSUB * RPW, LEFT_OUT)])

        pltpu.sync_copy(ones_hbm, ones_v)
        plsc.subcore_barrier()

        row_base = cid * DGRP_PC * GRP

        def do_group(row0, nrows):
            pltpu.sync_copy(dst_hbm.at[pl.ds(row0, nrows)],
                            dst_g.at[pl.ds(0, nrows)])
            hs = [pltpu.async_copy(ones_v, sdeg.at[dst_g.at[j]],
                                   ssem, add=True) for j in range(nrows)]
            for h in hs:
                h.wait()

        @pl.loop(0, (DGRP_PC + NSUB - 1) // NSUB)
        def _(i):
            g = sid + i * NSUB

            @pl.when(g < DGRP_PC)
            def _():
                do_group(row_base + g * GRP, GRP)

        @pl.when((cid == 1) & (sid == NSUB - 1))
        def _():
            do_group(2 * DGRP_PC * GRP, DLEFT)

        plsc.subcore_barrier()

        obase = cid * N + base
        pltpu.sync_copy(sdeg.at[pl.ds(base, RPW)],
                        out_hbm.at[pl.ds(obase, RPW)])

        @pl.when(sid == NSUB - 1)
        def _():
            pltpu.sync_copy(sdeg.at[pl.ds(NSUB * RPW, LEFT_OUT)],
                            out_hbm.at[pl.ds(cid * N + NSUB * RPW, LEFT_OUT)])

    return agg(dst2, zdeg, ones)


BLK = 400


def _tc_self(x, w_self, b_sage):
    def body(x_ref, ws_ref, bs_ref, o_ref):
        o_ref[...] = jnp.dot(
            x_ref[...].astype(jnp.bfloat16), ws_ref[...].astype(jnp.bfloat16),
            preferred_element_type=jnp.float32) + bs_ref[...]

    return pl.pallas_call(
        body,
        grid=(N // BLK,),
        in_specs=[
            pl.BlockSpec((BLK, IN_FEATS), lambda i: (i, 0)),
            pl.BlockSpec((IN_FEATS, HIDDEN), lambda i: (0, 0)),
            pl.BlockSpec((1, HIDDEN), lambda i: (0, 0)),
        ],
        out_specs=pl.BlockSpec((BLK, HIDDEN), lambda i: (i, 0)),
        out_shape=jax.ShapeDtypeStruct((N, HIDDEN), jnp.float32),
    )(x, w_self, b_sage)


def _tc_out(hs, sums, degs, wn0, wn1, w_fc, b_fc):
    def body(hs_ref, s0_ref, s1_ref, d0_ref, d1_ref, wn0_ref,
             wn1_ref, wfc_ref, bfc_ref, o_ref):
        deg = d0_ref[:, 0:1] + d1_ref[:, 0:1]
        inv = 1.0 / jnp.maximum(deg, 1.0)
        h = (
            hs_ref[...]
            + jnp.dot((s0_ref[...] * inv).astype(jnp.bfloat16),
                      wn0_ref[...].astype(jnp.bfloat16),
                      preferred_element_type=jnp.float32)
            + jnp.dot((s1_ref[...] * inv).astype(jnp.bfloat16),
                      wn1_ref[...].astype(jnp.bfloat16),
                      preferred_element_type=jnp.float32)
        )
        h = jnp.maximum(h, 0.0)
        o_ref[...] = (
            jnp.dot(h.astype(jnp.bfloat16), wfc_ref[...].astype(jnp.bfloat16),
                    preferred_element_type=jnp.float32)
            + bfc_ref[...]
        )

    nblk = N // BLK
    return pl.pallas_call(
        body,
        grid=(nblk,),
        in_specs=[
            pl.BlockSpec((BLK, HIDDEN), lambda i: (i, 0)),
            pl.BlockSpec((BLK, HALF), lambda i: (i, 0)),
            pl.BlockSpec((BLK, HALF), lambda i: (i + nblk, 0)),
            pl.BlockSpec((BLK, DEG_W), lambda i: (i, 0)),
            pl.BlockSpec((BLK, DEG_W), lambda i: (i + nblk, 0)),
            pl.BlockSpec((HALF, HIDDEN), lambda i: (0, 0)),
            pl.BlockSpec((HALF, HIDDEN), lambda i: (0, 0)),
            pl.BlockSpec((HIDDEN, NUM_CLASSES), lambda i: (0, 0)),
            pl.BlockSpec((1, NUM_CLASSES), lambda i: (0, 0)),
        ],
        out_specs=pl.BlockSpec((BLK, NUM_CLASSES), lambda i: (i, 0)),
        out_shape=jax.ShapeDtypeStruct((N, NUM_CLASSES), jnp.float32),
    )(hs, sums, sums, degs, degs, wn0, wn1, w_fc, b_fc)


def kernel(x, edge_index, W_self, W_neigh, b_sage, W_fc, b_fc):
    xv = x.reshape(2 * N, HALF)
    src2 = (edge_index[0] * 2).reshape(ROWS, CHUNK)
    dst2 = edge_index[1].reshape(ROWS, CHUNK)
    zsum = jnp.zeros((N, HALF), jnp.float32)
    zdeg = jnp.zeros((N, DEG_W), jnp.float32)
    ones = jnp.ones((CHUNK, DEG_W), jnp.float32)
    sums = _sc_sum(xv, src2, dst2, zsum)
    degs = _sc_deg(dst2, zdeg, ones)
    hs = _tc_self(x, W_self, b_sage.reshape(1, HIDDEN))
    return _tc_out(
        hs, sums, degs,
        W_neigh[:HALF], W_neigh[HALF:],
        W_fc, b_fc.reshape(1, NUM_CLASSES),
    )


# speedup vs baseline: 1.0136x; 1.0136x over previous
import functools

import jax
import jax.numpy as jnp
from jax import lax
from jax.experimental import pallas as pl
from jax.experimental.pallas import tpu as pltpu
from jax.experimental.pallas import tpu_sc as plsc

N = 10000
E = 160000
IN_FEATS = 256
HALF = 128
HIDDEN = 512
NUM_CLASSES = 128

CHUNK = 128
ROWS = E // CHUNK
NSUB = 16
LANES = 16
DEG_W = 128
ROWS_PC = ROWS // 2
RPW = (N // NSUB) // 8 * 8
LEFT_OUT = N - NSUB * RPW

_MESH = plsc.VectorSubcoreMesh(core_axis_name="c", subcore_axis_name="s")


GRP = 8
NGRP = ROWS // GRP
LEFT_ROWS = ROWS - NGRP * GRP
NB = 2
DEPTH = 2


def _sc_sum(xs, src2, dst2, zsum):
    @functools.partial(
        pl.kernel,
        out_type=jax.ShapeDtypeStruct((2 * N, HALF), jnp.float32),
        mesh=_MESH,
        scratch_types=[
            pltpu.VMEM((GRP, CHUNK), jnp.int32),
            pltpu.VMEM((GRP, CHUNK), jnp.int32),
            pltpu.VMEM((CHUNK, HALF), jnp.float32),
            pltpu.VMEM((CHUNK, HALF), jnp.float32),
            pltpu.SemaphoreType.DMA,
            pltpu.SemaphoreType.DMA,
            pltpu.SemaphoreType.DMA,
            pltpu.SemaphoreType.DMA,
            pltpu.VMEM_SHARED((N, HALF), jnp.float32),
        ],
    )
    def agg(xs_hbm, src_hbm, dst_hbm, zsum_hbm, out_hbm,
            src_g, dst_g, b0, b1, g0, g1, s0, s1, ssum):
        cid = lax.axis_index("c")
        sid = lax.axis_index("s")
        base = sid * RPW
        bufs = (b0, b1)
        gsem = (g0, g1)
        ssem = (s0, s1)

        pltpu.sync_copy(zsum_hbm.at[pl.ds(base, RPW)],
                        ssum.at[pl.ds(base, RPW)])

        @pl.when(sid == NSUB - 1)
        def _():
            pltpu.sync_copy(zsum_hbm.at[pl.ds(NSUB * RPW, LEFT_OUT)],
                            ssum.at[pl.ds(NSUB * RPW, LEFT_OUT)])

        plsc.subcore_barrier()

        off = cid

        def do_group(row0, nrows):
            pltpu.sync_copy(src_hbm.at[pl.ds(row0, nrows)],
                            src_g.at[pl.ds(0, nrows)])
            pltpu.sync_copy(dst_hbm.at[pl.ds(row0, nrows)],
                            dst_g.at[pl.ds(0, nrows)])
            for j in range(nrows):
                for k in range(CHUNK // LANES):
                    sl = pl.ds(k * LANES, LANES)
                    src_g[j, sl] = src_g[j, sl] + off

            gh = {}
            sh = {}
            waited = set()
            for j in range(min(DEPTH, nrows)):
                b = j % NB
                gh[j] = pltpu.async_copy(xs_hbm.at[src_g.at[j]],
                                         bufs[b], gsem[b])
            for j in range(nrows):
                b = j % NB
                gh[j].wait()
                sh[j] = pltpu.async_copy(bufs[b], ssum.at[dst_g.at[j]],
                                         ssem[b], add=True)
                nj = j + DEPTH
                if nj < nrows:
                    nb_ = nj % NB
                    if nj - NB >= 0:
                        sh[nj - NB].wait()
                        waited.add(nj - NB)
                    gh[nj] = pltpu.async_copy(xs_hbm.at[src_g.at[nj]],
                                              bufs[nb_], gsem[nb_])
            for j in range(nrows):
                if j not in waited:
                    sh[j].wait()

        @pl.loop(0, (NGRP + NSUB - 1) // NSUB)
        def _(i):
            g = sid + i * NSUB

            @pl.when(g < NGRP)
            def _():
                do_group(g * GRP, GRP)

        @pl.when(sid == NSUB - 1)
        def _():
            do_group(NGRP * GRP, LEFT_ROWS)

        plsc.subcore_barrier()

        out_off = cid * N
        pltpu.sync_copy(ssum.at[pl.ds(base, RPW)],
                        out_hbm.at[pl.ds(out_off + base, RPW)])

        @pl.when(sid == NSUB - 1)
        def _():
            pltpu.sync_copy(ssum.at[pl.ds(NSUB * RPW, LEFT_OUT)],
                            out_hbm.at[pl.ds(out_off + NSUB * RPW, LEFT_OUT)])

    return agg(xs, src2, dst2, zsum)


DGRP_PC = (ROWS // 2) // GRP
DLEFT = ROWS - 2 * DGRP_PC * GRP


def _sc_deg(dst2, zdeg, ones):
    @functools.partial(
        pl.kernel,
        out_type=jax.ShapeDtypeStruct((2 * N, DEG_W), jnp.float32),
        mesh=_MESH,
        scratch_types=[
            pltpu.VMEM((GRP, CHUNK), jnp.int32),
            pltpu.VMEM((CHUNK, DEG_W), jnp.float32),
            pltpu.SemaphoreType.DMA,
            pltpu.VMEM_SHARED((N, DEG_W), jnp.float32),
        ],
    )
    def agg(dst_hbm, zdeg_hbm, ones_hbm, out_hbm, dst_g, ones_v, ssem, sdeg):
        cid = lax.axis_index("c")
        sid = lax.axis_index("s")
        base = sid * RPW

        pltpu.sync_copy(zdeg_hbm.at[pl.ds(base, RPW)],
                        sdeg.at[pl.ds(base, RPW)])

        @pl.when(sid == NSUB - 1)
        def _():
            pltpu.sync_copy(zdeg_hbm.at[pl.ds(NSUB * RPW, LEFT_OUT)],
                            sdeg.at[pl.ds(NSUB * RPW, LEFT_OUT)])

        pltpu.sync_copy(ones_hbm, ones_v)
        plsc.subcore_barrier()

        row_base = cid * DGRP_PC * GRP

        def do_group(row0, nrows):
            pltpu.sync_copy(dst_hbm.at[pl.ds(row0, nrows)],
                            dst_g.at[pl.ds(0, nrows)])
            hs = [pltpu.async_copy(ones_v, sdeg.at[dst_g.at[j]],
                                   ssem, add=True) for j in range(nrows)]
            for h in hs:
                h.wait()

        @pl.loop(0, (DGRP_PC + NSUB - 1) // NSUB)
        def _(i):
            g = sid + i * NSUB

            @pl.when(g < DGRP_PC)
            def _():
                do_group(row_base + g * GRP, GRP)

        @pl.when((cid == 1) & (sid == NSUB - 1))
        def _():
            do_group(2 * DGRP_PC * GRP, DLEFT)

        plsc.subcore_barrier()

        obase = cid * N + base
        pltpu.sync_copy(sdeg.at[pl.ds(base, RPW)],
                        out_hbm.at[pl.ds(obase, RPW)])

        @pl.when(sid == NSUB - 1)
        def _():
            pltpu.sync_copy(sdeg.at[pl.ds(NSUB * RPW, LEFT_OUT)],
                            out_hbm.at[pl.ds(cid * N + NSUB * RPW, LEFT_OUT)])

    return agg(dst2, zdeg, ones)


BLK = 400


def _tc_dense(x, sums, degs, w_self, wn0, wn1, b_sage, w_fc, b_fc):
    def body(x_ref, s0_ref, s1_ref, d0_ref, d1_ref, ws_ref, wn0_ref,
             wn1_ref, bs_ref, wfc_ref, bfc_ref, o_ref):
        deg = d0_ref[:, 0:1] + d1_ref[:, 0:1]
        inv = 1.0 / jnp.maximum(deg, 1.0)
        h = (
            jnp.dot(x_ref[...].astype(jnp.bfloat16),
                    ws_ref[...].astype(jnp.bfloat16),
                    preferred_element_type=jnp.float32)
            + jnp.dot((s0_ref[...] * inv).astype(jnp.bfloat16),
                      wn0_ref[...].astype(jnp.bfloat16),
                      preferred_element_type=jnp.float32)
            + jnp.dot((s1_ref[...] * inv).astype(jnp.bfloat16),
                      wn1_ref[...].astype(jnp.bfloat16),
                      preferred_element_type=jnp.float32)
            + bs_ref[...]
        )
        h = jnp.maximum(h, 0.0)
        o_ref[...] = (
            jnp.dot(h.astype(jnp.bfloat16), wfc_ref[...].astype(jnp.bfloat16),
                    preferred_element_type=jnp.float32)
            + bfc_ref[...]
        )

    nblk = N // BLK
    return pl.pallas_call(
        body,
        grid=(nblk,),
        in_specs=[
            pl.BlockSpec((BLK, IN_FEATS), lambda i: (i, 0)),
            pl.BlockSpec((BLK, HALF), lambda i: (i, 0)),
            pl.BlockSpec((BLK, HALF), lambda i: (i + nblk, 0)),
            pl.BlockSpec((BLK, DEG_W), lambda i: (i, 0)),
            pl.BlockSpec((BLK, DEG_W), lambda i: (i + nblk, 0)),
            pl.BlockSpec((IN_FEATS, HIDDEN), lambda i: (0, 0)),
            pl.BlockSpec((HALF, HIDDEN), lambda i: (0, 0)),
            pl.BlockSpec((HALF, HIDDEN), lambda i: (0, 0)),
            pl.BlockSpec((1, HIDDEN), lambda i: (0, 0)),
            pl.BlockSpec((HIDDEN, NUM_CLASSES), lambda i: (0, 0)),
            pl.BlockSpec((1, NUM_CLASSES), lambda i: (0, 0)),
        ],
        out_specs=pl.BlockSpec((BLK, NUM_CLASSES), lambda i: (i, 0)),
        out_shape=jax.ShapeDtypeStruct((N, NUM_CLASSES), jnp.float32),
    )(x, sums, sums, degs, degs, w_self, wn0, wn1, b_sage, w_fc, b_fc)


def kernel(x, edge_index, W_self, W_neigh, b_sage, W_fc, b_fc):
    xv = x.reshape(2 * N, HALF)
    src2 = (edge_index[0] * 2).reshape(ROWS, CHUNK)
    dst2 = edge_index[1].reshape(ROWS, CHUNK)
    zsum = jnp.zeros((N, HALF), jnp.float32)
    zdeg = jnp.zeros((N, DEG_W), jnp.float32)
    ones = jnp.ones((CHUNK, DEG_W), jnp.float32)
    sums = _sc_sum(xv, src2, dst2, zsum)
    degs = _sc_deg(dst2, zdeg, ones)
    return _tc_dense(
        x, sums, degs,
        W_self, W_neigh[:HALF], W_neigh[HALF:],
        b_sage.reshape(1, HIDDEN), W_fc, b_fc.reshape(1, NUM_CLASSES),
    )
